# baseline (speedup 1.0000x reference)
import functools

import jax
import jax.numpy as jnp
from jax import lax
from jax.experimental import pallas as pl
from jax.experimental.pallas import tpu as pltpu
from jax.experimental.pallas import tpu_sc as plsc

N = 10000
NPAD = 10240
E = 320000
D = 128
DH = D // 2
NC, NS, L = 2, 16, 16
NW = NC * NS
B = 80
EPW = E // NW
NB = EPW // B
EPT = E // NS
NBT = EPT // B
RPT = NPAD // NS

_MESH = plsc.VectorSubcoreMesh(core_axis_name="c", subcore_axis_name="s")
_NO_LAYOUT = pltpu.CompilerParams(needs_layout_passes=False,
                                  use_tc_tiling_on_sc=False)


def _zero_fill_1d(ref, n):
    z = jnp.zeros((L,), jnp.float32)

    def body(i, _):
        ref[pl.ds(i * L, L)] = z
        return 0

    lax.fori_loop(0, n // L, body, 0)


@functools.partial(
    pl.kernel,
    out_type=jax.ShapeDtypeStruct((NC, 2, NPAD), jnp.float32),
    mesh=_MESH,
    compiler_params=_NO_LAYOUT,
    scratch_types=[
        pltpu.VMEM((NB, B), jnp.int32),
        pltpu.VMEM((NB, B), jnp.float32),
        pltpu.VMEM((B,), jnp.float32),
        pltpu.VMEM((RPT,), jnp.float32),
        pltpu.VMEM_SHARED((NPAD,), jnp.float32),
        pltpu.VMEM_SHARED((NPAD,), jnp.float32),
    ],
)
def _prep_deg_cnt(col_hbm, ew_hbm, out_hbm, col_v, ew_v, ones_v, zb_v,
                  deg_sh, cnt_sh):
    c = lax.axis_index("c")
    s = lax.axis_index("s")
    wid = c * NS + s
    one = jnp.ones((L,), jnp.float32)
    for j in range(B // L):
        ones_v[pl.ds(j * L, L)] = one
    _zero_fill_1d(zb_v, RPT)
    base = s * RPT
    pltpu.sync_copy(zb_v, deg_sh.at[pl.ds(base, RPT)])
    pltpu.sync_copy(zb_v, cnt_sh.at[pl.ds(base, RPT)])
    plsc.subcore_barrier()
    pltpu.sync_copy(col_hbm.at[wid], col_v)
    pltpu.sync_copy(ew_hbm.at[wid], ew_v)

    def body(i, _):
        idx = col_v.at[i]
        pltpu.sync_copy(ew_v.at[i], deg_sh.at[idx], add=True)
        pltpu.sync_copy(ones_v, cnt_sh.at[idx], add=True)
        return 0

    lax.fori_loop(0, NB, body, 0)
    plsc.subcore_barrier()
    pltpu.sync_copy(deg_sh.at[pl.ds(base, RPT)],
                    out_hbm.at[c, 0, pl.ds(base, RPT)])
    pltpu.sync_copy(cnt_sh.at[pl.ds(base, RPT)],
                    out_hbm.at[c, 1, pl.ds(base, RPT)])


def _ab_body(p_ref, ab_ref):
    deg = p_ref[0, 0, :] + p_ref[1, 0, :]
    cnt = p_ref[0, 1, :] + p_ref[1, 1, :]
    dinv = lax.rsqrt(deg)
    dinv = jnp.where(deg > 0.0, dinv, 0.0)
    ab_ref[0, :] = dinv / jnp.maximum(cnt, 1.0)
    ab_ref[1, :] = dinv


_ab_call = pl.pallas_call(
    _ab_body,
    out_shape=jax.ShapeDtypeStruct((2, NPAD), jnp.float32),
)


@functools.partial(
    pl.kernel,
    out_type=jax.ShapeDtypeStruct((NW, NB, B), jnp.float32),
    mesh=_MESH,
    compiler_params=_NO_LAYOUT,
    scratch_types=[
        pltpu.VMEM((NPAD,), jnp.float32),
        pltpu.VMEM((NPAD,), jnp.float32),
        pltpu.VMEM((NB, B), jnp.int32),
        pltpu.VMEM((NB, B), jnp.int32),
        pltpu.VMEM((NB, B), jnp.float32),
        pltpu.VMEM((NB, B), jnp.float32),
    ],
)
def _prep_w(ab_hbm, row_hbm, col_hbm, ew_hbm, w_hbm,
            a_v, b_v, row_v, col_v, ew_v, w_v):
    c = lax.axis_index("c")
    s = lax.axis_index("s")
    wid = c * NS + s
    pltpu.sync_copy(ab_hbm.at[0], a_v.at[pl.ds(0, NPAD)])
    pltpu.sync_copy(ab_hbm.at[1], b_v.at[pl.ds(0, NPAD)])
    pltpu.sync_copy(row_hbm.at[wid], row_v)
    pltpu.sync_copy(col_hbm.at[wid], col_v)
    pltpu.sync_copy(ew_hbm.at[wid], ew_v)

    def body(i, _):
        for j in range(B // L):
            sl = pl.ds(j * L, L)
            bg = plsc.load_gather(b_v, [row_v[i, sl]])
            ag = plsc.load_gather(a_v, [col_v[i, sl]])
            w_v[i, sl] = bg * ew_v[i, sl] * ag
        return 0

    lax.fori_loop(0, NB, body, 0)
    pltpu.sync_copy(w_v, w_hbm.at[wid])


NBUF = 4
NOB = 2
NBTP = NBT + 2


@functools.partial(
    pl.kernel,
    out_type=jax.ShapeDtypeStruct((NC, NPAD, DH), jnp.float32),
    mesh=_MESH,
    compiler_params=_NO_LAYOUT,
    scratch_types=[
        pltpu.VMEM((NBTP, B), jnp.int32),
        pltpu.VMEM((NBT, B), jnp.float32),
        pltpu.VMEM((NBUF, B), jnp.int32),
        pltpu.VMEM((NBUF, B), jnp.int32),
        [pltpu.VMEM((B, DH), jnp.bfloat16)] * NBUF,
        [pltpu.VMEM((B, DH), jnp.float32)] * NOB,
        pltpu.VMEM((128, DH), jnp.float32),
        pltpu.VMEM_SHARED((NPAD, DH), jnp.float32),
        [pltpu.SemaphoreType.DMA] * NBUF,
        pltpu.SemaphoreType.DMA,
    ],
)
def _agg(y_hbm, pk_hbm, w_hbm, out_hbm,
         pk_v, w_v, ri_v, ci_v, gbufs, obufs, zb_v, acc_sh, gsems, ssem):
    c = lax.axis_index("c")
    s = lax.axis_index("s")
    z = jnp.zeros((L,), jnp.float32)

    def zrow(i, _):
        for f in range(DH // L):
            zb_v[i, pl.ds(f * L, L)] = z
        return 0

    lax.fori_loop(0, 128, zrow, 0)
    base = s * RPT
    for k in range(RPT // 128):
        pltpu.sync_copy(zb_v, acc_sh.at[pl.ds(base + k * 128, 128), :])
    plsc.subcore_barrier()
    pltpu.sync_copy(pk_hbm.at[s], pk_v)
    pltpu.sync_copy(w_hbm.at[s], w_v)

    def make_idx(i, u):
        for j in range(B // L):
            sl = pl.ds(j * L, L)
            p16 = pk_v[i, sl]
            ri_v[u, sl] = p16 & 0xFFFF
            ci_v[u, sl] = lax.shift_right_logical(p16, 16)

    def scale(i, g, o):
        def jbody(j, _):
            w16 = w_v[i, pl.ds(j * L, L)]
            for e in range(L):
                wv = w16[e]
                ebase = j * L + e
                vals = []
                for jj in range(DH // (2 * L)):
                    chunk = g[ebase, pl.ds(2 * L * jj, 2 * L)]
                    va, vb = plsc.unpack(chunk,
                                         format=plsc.PackFormat.INTERLEAVED)
                    vals.append(va)
                    vals.append(vb)
                for f in range(DH // L):
                    o[ebase, pl.ds(f * L, L)] = vals[f] * wv
            return 0

        lax.fori_loop(0, B // L, jbody, 0)

    def issue_gather(u):
        pltpu.async_copy(y_hbm.at[c].at[ri_v.at[u]], gbufs[u], gsems[u])

    def wait_gather(u):
        pltpu.make_async_copy(y_hbm.at[c, pl.ds(0, B), :], gbufs[u],
                              gsems[u]).wait()

    def issue_scatter(u, uo):
        pltpu.async_copy(obufs[uo], acc_sh.at[ci_v.at[u]], ssem, add=True)

    def wait_scatter():
        pltpu.make_async_copy(obufs[0], acc_sh.at[pl.ds(0, B), :],
                              ssem).wait()

    for u in range(NBUF):
        make_idx(u, u)
        issue_gather(u)
    for i in range(2):
        wait_gather(i)
        scale(i, gbufs[i], obufs[i])
        issue_scatter(i, i)

    def round_body(k, _):
        ib = 4 * k + 2
        for u0 in range(NBUF):
            i = ib + u0
            u = (2 + u0) % NBUF
            wait_gather(u)
            wait_scatter()
            make_idx(i + 2, u0)
            issue_gather(u0)
            scale(i, gbufs[u], obufs[u0 % NOB])
            issue_scatter(u, u0 % NOB)
        return 0

    lax.fori_loop(0, (NBT - 2) // NBUF, round_body, 0)

    wait_gather(2)
    wait_gather(3)
    wait_scatter()
    wait_scatter()
    plsc.subcore_barrier()
    pltpu.sync_copy(acc_sh.at[pl.ds(base, RPT), :],
                    out_hbm.at[c, pl.ds(base, RPT), :])


def _mm_body(p_ref, y_ref, wlT_ref, bl_ref, wrT_ref, o_ref, *, relu):
    acc = jnp.dot(p_ref[0], wlT_ref[0], preferred_element_type=jnp.float32)
    acc = acc + jnp.dot(p_ref[1], wlT_ref[1],
                        preferred_element_type=jnp.float32)
    acc = acc + bl_ref[...]
    acc = acc + jnp.dot(y_ref[...], wrT_ref[...],
                        preferred_element_type=jnp.float32)
    if relu:
        acc = jnp.maximum(acc, 0.0)
    o_ref[...] = acc


def _mm(P, y, WlT, bl, WrT, relu):
    R = 1000
    return pl.pallas_call(
        functools.partial(_mm_body, relu=relu),
        grid=(N // R,),
        in_specs=[
            pl.BlockSpec((2, R, DH), lambda i: (0, i, 0)),
            pl.BlockSpec((R, D), lambda i: (i, 0)),
            pl.BlockSpec((2, DH, D), lambda i: (0, 0, 0)),
            pl.BlockSpec((1, D), lambda i: (0, 0)),
            pl.BlockSpec((D, D), lambda i: (0, 0)),
        ],
        out_specs=pl.BlockSpec((R, D), lambda i: (i, 0)),
        out_shape=jax.ShapeDtypeStruct((N, D), jnp.float32),
    )(P, y, WlT, bl, WrT)


_HPERM = sum([[g + k for k in range(0, 32, 2)] + [g + k for k in range(1, 32, 2)]
              for g in range(0, DH, 32)], [])


def kernel(x, edge_index, edge_weight, W1l, b1l, W1r, W2l, b2l, W2r):
    row3 = edge_index[0].reshape(NW, NB, B)
    col3 = edge_index[1].reshape(NW, NB, B)
    ew3 = edge_weight.reshape(NW, NB, B)
    packed = edge_index[0] | (edge_index[1] << 16)
    pk3t = jnp.concatenate(
        [packed.reshape(NS, NBT, B),
         jnp.zeros((NS, NBTP - NBT, B), jnp.int32)], axis=1)

    dc = _prep_deg_cnt(col3, ew3)
    ab = _ab_call(dc)
    w3 = _prep_w(ab, row3, col3, ew3)
    w3t = w3.reshape(NS, NBT, B)

    def layer(y, Wl, bl, Wr, relu):
        ysp = y.reshape(N, 2, DH).transpose(1, 0, 2).astype(jnp.bfloat16)
        p = _agg(ysp, pk3t, w3t)
        wlt = Wl.T.reshape(2, DH, D)[:, _HPERM, :]
        return _mm(p, y, wlt, bl.reshape(1, D), Wr.T, relu)

    h = layer(x, W1l, b1l, W1r, relu=True)
    out = layer(h, W2l, b2l, W2r, relu=False)
    return out


# speedup vs baseline: 1.0830x; 1.0830x over previous
import functools

import jax
import jax.numpy as jnp
from jax import lax
from jax.experimental import pallas as pl
from jax.experimental.pallas import tpu as pltpu
from jax.experimental.pallas import tpu_sc as plsc

N = 10000
NPAD = 10240
E = 320000
D = 128
DH = D // 2
NC, NS, L = 2, 16, 16
NW = NC * NS
B = 80
EPW = E // NW
NB = EPW // B
EPT = E // NS
NBT = EPT // B
RPT = NPAD // NS

_MESH = plsc.VectorSubcoreMesh(core_axis_name="c", subcore_axis_name="s")
_NO_LAYOUT = pltpu.CompilerParams(needs_layout_passes=False,
                                  use_tc_tiling_on_sc=False)


def _zero_fill_1d(ref, n):
    z = jnp.zeros((L,), jnp.float32)

    def body(i, _):
        ref[pl.ds(i * L, L)] = z
        return 0

    lax.fori_loop(0, n // L, body, 0)


@functools.partial(
    pl.kernel,
    out_type=jax.ShapeDtypeStruct((NC, 2, NPAD), jnp.float32),
    mesh=_MESH,
    compiler_params=_NO_LAYOUT,
    scratch_types=[
        pltpu.VMEM((NB, B), jnp.int32),
        pltpu.VMEM((NB, B), jnp.float32),
        pltpu.VMEM((B,), jnp.float32),
        pltpu.VMEM((RPT,), jnp.float32),
        pltpu.VMEM_SHARED((NPAD,), jnp.float32),
        pltpu.VMEM_SHARED((NPAD,), jnp.float32),
    ],
)
def _prep_deg_cnt(col_hbm, ew_hbm, out_hbm, col_v, ew_v, ones_v, zb_v,
                  deg_sh, cnt_sh):
    c = lax.axis_index("c")
    s = lax.axis_index("s")
    wid = c * NS + s
    one = jnp.ones((L,), jnp.float32)
    for j in range(B // L):
        ones_v[pl.ds(j * L, L)] = one
    _zero_fill_1d(zb_v, RPT)
    base = s * RPT
    pltpu.sync_copy(zb_v, deg_sh.at[pl.ds(base, RPT)])
    pltpu.sync_copy(zb_v, cnt_sh.at[pl.ds(base, RPT)])
    plsc.subcore_barrier()
    pltpu.sync_copy(col_hbm.at[wid], col_v)
    pltpu.sync_copy(ew_hbm.at[wid], ew_v)

    def body(i, _):
        idx = col_v.at[i]
        pltpu.sync_copy(ew_v.at[i], deg_sh.at[idx], add=True)
        pltpu.sync_copy(ones_v, cnt_sh.at[idx], add=True)
        return 0

    lax.fori_loop(0, NB, body, 0)
    plsc.subcore_barrier()
    pltpu.sync_copy(deg_sh.at[pl.ds(base, RPT)],
                    out_hbm.at[c, 0, pl.ds(base, RPT)])
    pltpu.sync_copy(cnt_sh.at[pl.ds(base, RPT)],
                    out_hbm.at[c, 1, pl.ds(base, RPT)])


def _ab_body(p_ref, ab_ref):
    deg = p_ref[0, 0, :] + p_ref[1, 0, :]
    cnt = p_ref[0, 1, :] + p_ref[1, 1, :]
    dinv = lax.rsqrt(deg)
    dinv = jnp.where(deg > 0.0, dinv, 0.0)
    ab_ref[0, :] = dinv / jnp.maximum(cnt, 1.0)
    ab_ref[1, :] = dinv


_ab_call = pl.pallas_call(
    _ab_body,
    out_shape=jax.ShapeDtypeStruct((2, NPAD), jnp.float32),
)


@functools.partial(
    pl.kernel,
    out_type=jax.ShapeDtypeStruct((NW, NB, B), jnp.float32),
    mesh=_MESH,
    compiler_params=_NO_LAYOUT,
    scratch_types=[
        pltpu.VMEM((NPAD,), jnp.float32),
        pltpu.VMEM((NPAD,), jnp.float32),
        pltpu.VMEM((NB, B), jnp.int32),
        pltpu.VMEM((NB, B), jnp.int32),
        pltpu.VMEM((NB, B), jnp.float32),
        pltpu.VMEM((NB, B), jnp.float32),
    ],
)
def _prep_w(ab_hbm, row_hbm, col_hbm, ew_hbm, w_hbm,
            a_v, b_v, row_v, col_v, ew_v, w_v):
    c = lax.axis_index("c")
    s = lax.axis_index("s")
    wid = c * NS + s
    pltpu.sync_copy(ab_hbm.at[0], a_v.at[pl.ds(0, NPAD)])
    pltpu.sync_copy(ab_hbm.at[1], b_v.at[pl.ds(0, NPAD)])
    pltpu.sync_copy(row_hbm.at[wid], row_v)
    pltpu.sync_copy(col_hbm.at[wid], col_v)
    pltpu.sync_copy(ew_hbm.at[wid], ew_v)

    def body(i, _):
        for j in range(B // L):
            sl = pl.ds(j * L, L)
            bg = plsc.load_gather(b_v, [row_v[i, sl]])
            ag = plsc.load_gather(a_v, [col_v[i, sl]])
            w_v[i, sl] = bg * ew_v[i, sl] * ag
        return 0

    lax.fori_loop(0, NB, body, 0)
    pltpu.sync_copy(w_v, w_hbm.at[wid])


NBUF = 4
NOB = 2
NBTP = NBT + 2


@functools.partial(
    pl.kernel,
    out_type=jax.ShapeDtypeStruct((NC, NPAD, DH), jnp.float32),
    mesh=_MESH,
    compiler_params=_NO_LAYOUT,
    scratch_types=[
        pltpu.VMEM((NBTP, B), jnp.int32),
        pltpu.VMEM((NBT, B), jnp.float32),
        pltpu.VMEM((NBUF, B), jnp.int32),
        pltpu.VMEM((NBUF, B), jnp.int32),
        [pltpu.VMEM((B, DH), jnp.float32)] * NBUF,
        [pltpu.VMEM((B, DH), jnp.float32)] * NOB,
        pltpu.VMEM((128, DH), jnp.float32),
        pltpu.VMEM_SHARED((NPAD, DH), jnp.float32),
        [pltpu.SemaphoreType.DMA] * NBUF,
        pltpu.SemaphoreType.DMA,
    ],
)
def _agg(y_hbm, pk_hbm, w_hbm, out_hbm,
         pk_v, w_v, ri_v, ci_v, gbufs, obufs, zb_v, acc_sh, gsems, ssem):
    c = lax.axis_index("c")
    s = lax.axis_index("s")
    z = jnp.zeros((L,), jnp.float32)

    def zrow(i, _):
        for f in range(DH // L):
            zb_v[i, pl.ds(f * L, L)] = z
        return 0

    lax.fori_loop(0, 128, zrow, 0)
    base = s * RPT
    for k in range(RPT // 128):
        pltpu.sync_copy(zb_v, acc_sh.at[pl.ds(base + k * 128, 128), :])
    plsc.subcore_barrier()
    pltpu.sync_copy(pk_hbm.at[s], pk_v)
    pltpu.sync_copy(w_hbm.at[s], w_v)

    def make_idx(i, u):
        for j in range(B // L):
            sl = pl.ds(j * L, L)
            p16 = pk_v[i, sl]
            ri_v[u, sl] = p16 & 0xFFFF
            ci_v[u, sl] = lax.shift_right_logical(p16, 16)

    def scale(i, g, o):
        def jbody(j, _):
            w16 = w_v[i, pl.ds(j * L, L)]
            for e in range(L):
                wv = w16[e]
                ebase = j * L + e
                vals = [g[ebase, pl.ds(f * L, L)] for f in range(DH // L)]
                for f in range(DH // L):
                    o[ebase, pl.ds(f * L, L)] = vals[f] * wv
            return 0

        lax.fori_loop(0, B // L, jbody, 0)

    H = B // 2

    def issue_gather(u):
        pltpu.async_copy(y_hbm.at[c].at[ri_v.at[u, pl.ds(0, H)]],
                         gbufs[u].at[pl.ds(0, H), :], gsems[u])
        pltpu.async_copy(y_hbm.at[c].at[ri_v.at[u, pl.ds(H, H)]],
                         gbufs[u].at[pl.ds(H, H), :], gsems[u])

    def wait_gather(u):
        pltpu.make_async_copy(y_hbm.at[c, pl.ds(0, B), :], gbufs[u],
                              gsems[u]).wait()

    def issue_scatter(u, uo):
        pltpu.async_copy(obufs[uo], acc_sh.at[ci_v.at[u]], ssem, add=True)

    def wait_scatter():
        pltpu.make_async_copy(obufs[0], acc_sh.at[pl.ds(0, B), :],
                              ssem).wait()

    for u in range(NBUF):
        make_idx(u, u)
        issue_gather(u)
    for i in range(2):
        wait_gather(i)
        scale(i, gbufs[i], obufs[i])
        issue_scatter(i, i)

    def round_body(k, _):
        ib = 4 * k + 2
        for u0 in range(NBUF):
            i = ib + u0
            u = (2 + u0) % NBUF
            wait_gather(u)
            wait_scatter()
            make_idx(i + 2, u0)
            issue_gather(u0)
            scale(i, gbufs[u], obufs[u0 % NOB])
            issue_scatter(u, u0 % NOB)
        return 0

    lax.fori_loop(0, (NBT - 2) // NBUF, round_body, 0)

    wait_gather(2)
    wait_gather(3)
    wait_scatter()
    wait_scatter()
    plsc.subcore_barrier()
    pltpu.sync_copy(acc_sh.at[pl.ds(base, RPT), :],
                    out_hbm.at[c, pl.ds(base, RPT), :])


def _mm_body(p_ref, y_ref, wlT_ref, bl_ref, wrT_ref, o_ref, *, relu):
    acc = jnp.dot(p_ref[0], wlT_ref[0], preferred_element_type=jnp.float32)
    acc = acc + jnp.dot(p_ref[1], wlT_ref[1],
                        preferred_element_type=jnp.float32)
    acc = acc + bl_ref[...]
    acc = acc + jnp.dot(y_ref[...], wrT_ref[...],
                        preferred_element_type=jnp.float32)
    if relu:
        acc = jnp.maximum(acc, 0.0)
    o_ref[...] = acc


def _mm(P, y, WlT, bl, WrT, relu):
    R = 1000
    return pl.pallas_call(
        functools.partial(_mm_body, relu=relu),
        grid=(N // R,),
        in_specs=[
            pl.BlockSpec((2, R, DH), lambda i: (0, i, 0)),
            pl.BlockSpec((R, D), lambda i: (i, 0)),
            pl.BlockSpec((2, DH, D), lambda i: (0, 0, 0)),
            pl.BlockSpec((1, D), lambda i: (0, 0)),
            pl.BlockSpec((D, D), lambda i: (0, 0)),
        ],
        out_specs=pl.BlockSpec((R, D), lambda i: (i, 0)),
        out_shape=jax.ShapeDtypeStruct((N, D), jnp.float32),
    )(P, y, WlT, bl, WrT)


def kernel(x, edge_index, edge_weight, W1l, b1l, W1r, W2l, b2l, W2r):
    row3 = edge_index[0].reshape(NW, NB, B)
    col3 = edge_index[1].reshape(NW, NB, B)
    ew3 = edge_weight.reshape(NW, NB, B)
    packed = edge_index[0] | (edge_index[1] << 16)
    pk3t = jnp.concatenate(
        [packed.reshape(NS, NBT, B),
         jnp.zeros((NS, NBTP - NBT, B), jnp.int32)], axis=1)

    dc = _prep_deg_cnt(col3, ew3)
    ab = _ab_call(dc)
    w3 = _prep_w(ab, row3, col3, ew3)
    w3t = w3.reshape(NS, NBT, B)

    def layer(y, Wl, bl, Wr, relu):
        ysp = y.reshape(N, 2, DH).transpose(1, 0, 2)
        p = _agg(ysp, pk3t, w3t)
        return _mm(p, y, Wl.T.reshape(2, DH, D), bl.reshape(1, D),
                   Wr.T, relu)

    h = layer(x, W1l, b1l, W1r, relu=True)
    out = layer(h, W2l, b2l, W2r, relu=False)
    return out


# speedup vs baseline: 1.1303x; 1.0437x over previous
import functools

import jax
import jax.numpy as jnp
from jax import lax
from jax.experimental import pallas as pl
from jax.experimental.pallas import tpu as pltpu
from jax.experimental.pallas import tpu_sc as plsc

N = 10000
NPAD = 10240
E = 320000
D = 128
DH = D // 2
NC, NS, L = 2, 16, 16
NW = NC * NS
B = 80
EPW = E // NW
NB = EPW // B
EPT = E // NS
NBT = EPT // B
RPT = NPAD // NS

_MESH = plsc.VectorSubcoreMesh(core_axis_name="c", subcore_axis_name="s")
_NO_LAYOUT = pltpu.CompilerParams(needs_layout_passes=False,
                                  use_tc_tiling_on_sc=False)


def _zero_fill_1d(ref, n):
    z = jnp.zeros((L,), jnp.float32)

    def body(i, _):
        ref[pl.ds(i * L, L)] = z
        return 0

    lax.fori_loop(0, n // L, body, 0)


@functools.partial(
    pl.kernel,
    out_type=jax.ShapeDtypeStruct((NS, NBT, B), jnp.float32),
    mesh=_MESH,
    compiler_params=_NO_LAYOUT,
    scratch_types=[
        pltpu.VMEM((NBT, B), jnp.int32),
        pltpu.VMEM((NBT, B), jnp.float32),
        pltpu.VMEM((NBT, B), jnp.int32),
        pltpu.VMEM((NBT, B), jnp.float32),
        pltpu.VMEM((NPAD,), jnp.float32),
        pltpu.VMEM((NPAD,), jnp.float32),
        pltpu.VMEM((RPT,), jnp.float32),
        pltpu.VMEM((RPT,), jnp.float32),
        pltpu.VMEM((RPT,), jnp.float32),
        pltpu.VMEM((RPT,), jnp.float32),
        pltpu.VMEM((B,), jnp.float32),
        pltpu.VMEM_SHARED((NPAD,), jnp.float32),
        pltpu.VMEM_SHARED((NPAD,), jnp.float32),
        pltpu.VMEM_SHARED((NPAD,), jnp.float32),
        pltpu.VMEM_SHARED((NPAD,), jnp.float32),
        pltpu.SemaphoreType.DMA,
    ],
)
def _prep_all(pk_hbm, ew_hbm, w_hbm,
              pk_v, ew_v, col_v, w_v, a_v, b_v, dl_v, cl_v, al_v, bl_v,
              ones_v, deg_sh, cnt_sh, a_sh, b_sh, ssem):
    s = lax.axis_index("s")
    one = jnp.ones((L,), jnp.float32)
    for j in range(B // L):
        ones_v[pl.ds(j * L, L)] = one
    _zero_fill_1d(dl_v, RPT)
    base = s * RPT
    pltpu.sync_copy(dl_v, deg_sh.at[pl.ds(base, RPT)])
    pltpu.sync_copy(dl_v, cnt_sh.at[pl.ds(base, RPT)])
    plsc.subcore_barrier()
    pltpu.sync_copy(pk_hbm.at[s, pl.ds(0, NBT), :], pk_v)
    pltpu.sync_copy(ew_hbm.at[s], ew_v)

    def extract(i, _):
        for j in range(B // L):
            sl = pl.ds(j * L, L)
            col_v[i, sl] = lax.shift_right_logical(pk_v[i, sl], 16)
        return 0

    lax.fori_loop(0, NBT, extract, 0)

    def scat(i, _):
        idx = col_v.at[i]
        pltpu.async_copy(ew_v.at[i], deg_sh.at[idx], ssem, add=True)
        pltpu.async_copy(ones_v, cnt_sh.at[idx], ssem, add=True)
        return 0

    lax.fori_loop(0, NBT, scat, 0)

    def drain(i, _):
        pltpu.make_async_copy(ew_v.at[0], deg_sh.at[pl.ds(0, B)],
                              ssem).wait()
        pltpu.make_async_copy(ew_v.at[0], cnt_sh.at[pl.ds(0, B)],
                              ssem).wait()
        return 0

    lax.fori_loop(0, NBT, drain, 0)
    plsc.subcore_barrier()
    pltpu.sync_copy(deg_sh.at[pl.ds(base, RPT)], dl_v)
    pltpu.sync_copy(cnt_sh.at[pl.ds(base, RPT)], cl_v)

    def abloop(i, _):
        sl = pl.ds(i * L, L)
        dg = dl_v[sl]
        ct = cl_v[sl]
        u = plsc.bitcast(dg, jnp.int32)
        m = jnp.int32(0x5F3759DF) - lax.shift_right_logical(u, 1)
        r = plsc.bitcast(m, jnp.float32)
        hs = 0.5 * dg
        for _unused in range(3):
            r = r * (1.5 - hs * r * r)
        dinv = jnp.where(dg > 0.0, r, 0.0)
        al_v[sl] = dinv / jnp.maximum(ct, 1.0)
        bl_v[sl] = dinv
        return 0

    lax.fori_loop(0, RPT // L, abloop, 0)
    pltpu.sync_copy(al_v, a_sh.at[pl.ds(base, RPT)])
    pltpu.sync_copy(bl_v, b_sh.at[pl.ds(base, RPT)])
    plsc.subcore_barrier()
    pltpu.sync_copy(a_sh, a_v)
    pltpu.sync_copy(b_sh, b_v)

    def wloop(i, _):
        for j in range(B // L):
            sl = pl.ds(j * L, L)
            p16 = pk_v[i, sl]
            r16 = p16 & 0xFFFF
            bg = plsc.load_gather(b_v, [r16])
            ag = plsc.load_gather(a_v, [col_v[i, sl]])
            w_v[i, sl] = bg * ew_v[i, sl] * ag
        return 0

    lax.fori_loop(0, NBT, wloop, 0)
    pltpu.sync_copy(w_v, w_hbm.at[s])


NBUF = 4
NOB = 2
NBTP = NBT + 2


@functools.partial(
    pl.kernel,
    out_type=jax.ShapeDtypeStruct((NC, NPAD, DH), jnp.float32),
    mesh=_MESH,
    compiler_params=_NO_LAYOUT,
    scratch_types=[
        pltpu.VMEM((NBTP, B), jnp.int32),
        pltpu.VMEM((NBT, B), jnp.float32),
        pltpu.VMEM((NBUF, B), jnp.int32),
        pltpu.VMEM((NBUF, B), jnp.int32),
        [pltpu.VMEM((B, DH), jnp.float32)] * NBUF,
        [pltpu.VMEM((B, DH), jnp.float32)] * NOB,
        pltpu.VMEM((128, DH), jnp.float32),
        pltpu.VMEM_SHARED((NPAD, DH), jnp.float32),
        [pltpu.SemaphoreType.DMA] * NBUF,
        pltpu.SemaphoreType.DMA,
    ],
)
def _agg(y_hbm, pk_hbm, w_hbm, out_hbm,
         pk_v, w_v, ri_v, ci_v, gbufs, obufs, zb_v, acc_sh, gsems, ssem):
    c = lax.axis_index("c")
    s = lax.axis_index("s")
    z = jnp.zeros((L,), jnp.float32)

    def zrow(i, _):
        for f in range(DH // L):
            zb_v[i, pl.ds(f * L, L)] = z
        return 0

    lax.fori_loop(0, 128, zrow, 0)
    base = s * RPT
    for k in range(RPT // 128):
        pltpu.sync_copy(zb_v, acc_sh.at[pl.ds(base + k * 128, 128), :])
    plsc.subcore_barrier()
    pltpu.sync_copy(pk_hbm.at[s], pk_v)
    pltpu.sync_copy(w_hbm.at[s], w_v)

    def make_idx(i, u):
        for j in range(B // L):
            sl = pl.ds(j * L, L)
            p16 = pk_v[i, sl]
            ri_v[u, sl] = p16 & 0xFFFF
            ci_v[u, sl] = lax.shift_right_logical(p16, 16)

    def scale(i, g, o):
        def jbody(j, _):
            w16 = w_v[i, pl.ds(j * L, L)]
            for e in range(L):
                wv = w16[e]
                ebase = j * L + e
                vals = [g[ebase, pl.ds(f * L, L)] for f in range(DH // L)]
                for f in range(DH // L):
                    o[ebase, pl.ds(f * L, L)] = vals[f] * wv
            return 0

        lax.fori_loop(0, B // L, jbody, 0)

    H = B // 2

    def issue_gather(u):
        pltpu.async_copy(y_hbm.at[c].at[ri_v.at[u, pl.ds(0, H)]],
                         gbufs[u].at[pl.ds(0, H), :], gsems[u])
        pltpu.async_copy(y_hbm.at[c].at[ri_v.at[u, pl.ds(H, H)]],
                         gbufs[u].at[pl.ds(H, H), :], gsems[u])

    def wait_gather(u):
        pltpu.make_async_copy(y_hbm.at[c, pl.ds(0, B), :], gbufs[u],
                              gsems[u]).wait()

    def issue_scatter(u, uo):
        pltpu.async_copy(obufs[uo], acc_sh.at[ci_v.at[u]], ssem, add=True)

    def wait_scatter():
        pltpu.make_async_copy(obufs[0], acc_sh.at[pl.ds(0, B), :],
                              ssem).wait()

    for u in range(NBUF):
        make_idx(u, u)
        issue_gather(u)
    for i in range(2):
        wait_gather(i)
        scale(i, gbufs[i], obufs[i])
        issue_scatter(i, i)

    def round_body(k, _):
        ib = 4 * k + 2
        for u0 in range(NBUF):
            i = ib + u0
            u = (2 + u0) % NBUF
            wait_gather(u)
            wait_scatter()
            make_idx(i + 2, u0)
            issue_gather(u0)
            scale(i, gbufs[u], obufs[u0 % NOB])
            issue_scatter(u, u0 % NOB)
        return 0

    lax.fori_loop(0, (NBT - 2) // NBUF, round_body, 0)

    wait_gather(2)
    wait_gather(3)
    wait_scatter()
    wait_scatter()
    plsc.subcore_barrier()
    pltpu.sync_copy(acc_sh.at[pl.ds(base, RPT), :],
                    out_hbm.at[c, pl.ds(base, RPT), :])


def _mm_body(p_ref, y_ref, wlT_ref, bl_ref, wrT_ref, o_ref, *, relu):
    acc = jnp.dot(p_ref[0], wlT_ref[0], preferred_element_type=jnp.float32)
    acc = acc + jnp.dot(p_ref[1], wlT_ref[1],
                        preferred_element_type=jnp.float32)
    acc = acc + bl_ref[...]
    acc = acc + jnp.dot(y_ref[...], wrT_ref[...],
                        preferred_element_type=jnp.float32)
    if relu:
        acc = jnp.maximum(acc, 0.0)
    o_ref[...] = acc


def _mm(P, y, WlT, bl, WrT, relu):
    R = 1000
    return pl.pallas_call(
        functools.partial(_mm_body, relu=relu),
        grid=(N // R,),
        in_specs=[
            pl.BlockSpec((2, R, DH), lambda i: (0, i, 0)),
            pl.BlockSpec((R, D), lambda i: (i, 0)),
            pl.BlockSpec((2, DH, D), lambda i: (0, 0, 0)),
            pl.BlockSpec((1, D), lambda i: (0, 0)),
            pl.BlockSpec((D, D), lambda i: (0, 0)),
        ],
        out_specs=pl.BlockSpec((R, D), lambda i: (i, 0)),
        out_shape=jax.ShapeDtypeStruct((N, D), jnp.float32),
    )(P, y, WlT, bl, WrT)


def kernel(x, edge_index, edge_weight, W1l, b1l, W1r, W2l, b2l, W2r):
    packed = edge_index[0] | (edge_index[1] << 16)
    pk3t = jnp.concatenate(
        [packed.reshape(NS, NBT, B),
         jnp.zeros((NS, NBTP - NBT, B), jnp.int32)], axis=1)
    ew3t = edge_weight.reshape(NS, NBT, B)

    w3t = _prep_all(pk3t, ew3t)

    def layer(y, Wl, bl, Wr, relu):
        ysp = y.reshape(N, 2, DH).transpose(1, 0, 2)
        p = _agg(ysp, pk3t, w3t)
        return _mm(p, y, Wl.T.reshape(2, DH, D), bl.reshape(1, D),
                   Wr.T, relu)

    h = layer(x, W1l, b1l, W1r, relu=True)
    out = layer(h, W2l, b2l, W2r, relu=False)
    return out
